# Initial kernel scaffold; baseline (speedup 1.0000x reference)
import jax
import jax.numpy as jnp
from jax.experimental import pallas as pl


def kernel(node_type, in_degree, out_degree, node_weight, in_degree_weight, out_degree_weight):
    raise NotImplementedError("write your pallas kernel here")


# speedup vs baseline: 3.9067x; 3.9067x over previous
import functools

import jax
import jax.numpy as jnp
from jax import lax
from jax.experimental import pallas as pl
from jax.experimental.pallas import tpu as pltpu
from jax.experimental.pallas import tpu_sc as plsc

N_NODES = 100000
D = 128
NT = 513
NI = 256
K = 10

NC = 2
NS = 16
NW = NC * NS

B = 32
SUB = 8
G = B * SUB
NG = (N_NODES + G - 1) // G
N_PAD = NG * G
ROUNDS = (NG + NW - 1) // NW


def _sc_body(table_hbm, idx_hbm, out_hbm, idx_v, rows_v, out_v, gsem):
    wid = lax.axis_index("s") * NC + lax.axis_index("c")

    def group_body(r, carry):
        g = wid + r * NW

        @pl.when(g < NG)
        def _():
            base = g * G
            for k in range(K):
                pltpu.sync_copy(idx_hbm.at[pl.ds(k * N_PAD + base, G)],
                                idx_v.at[k])

            def sub_body(s, c):
                sbase = base + s * B
                for k in range(K):
                    pltpu.async_copy(
                        table_hbm.at[idx_v.at[k, pl.ds(s * B, B)]],
                        rows_v.at[k], gsem)
                for k in range(K):
                    pltpu.make_async_copy(
                        table_hbm.at[idx_v.at[k, pl.ds(s * B, B)]],
                        rows_v.at[k], gsem).wait()

                def node_body(b, cc):
                    for j in range(D // 16):
                        acc = rows_v[0, b, pl.ds(j * 16, 16)]
                        for k in range(1, K):
                            acc = acc + rows_v[k, b, pl.ds(j * 16, 16)]
                        out_v[b, pl.ds(j * 16, 16)] = acc
                    return cc

                lax.fori_loop(0, B, node_body, 0)
                pltpu.sync_copy(out_v, out_hbm.at[pl.ds(sbase, B)])
                return c

            lax.fori_loop(0, SUB, sub_body, 0)

        return carry

    lax.fori_loop(0, ROUNDS, group_body, 0)


def kernel(node_type, in_degree, out_degree, node_weight, in_degree_weight,
           out_degree_weight):
    table = jnp.concatenate(
        [node_weight, in_degree_weight, out_degree_weight], axis=0)
    idx = jnp.concatenate(
        [node_type.astype(jnp.int32),
         (in_degree.astype(jnp.int32) + NT)[:, None],
         (out_degree.astype(jnp.int32) + NT + NI)[:, None]], axis=1).T
    idx = jnp.pad(idx, ((0, 0), (0, N_PAD - N_NODES))).reshape(-1)

    mesh = plsc.VectorSubcoreMesh(core_axis_name="c", subcore_axis_name="s",
                                  num_cores=NC, num_subcores=NS)
    run = functools.partial(
        pl.kernel,
        out_type=jax.ShapeDtypeStruct((N_PAD, D), jnp.float32),
        mesh=mesh,
        scratch_types=[
            pltpu.VMEM((K, G), jnp.int32),
            pltpu.VMEM((K, B, D), jnp.float32),
            pltpu.VMEM((B, D), jnp.float32),
            pltpu.SemaphoreType.DMA,
        ],
    )(_sc_body)
    out = run(table, idx)
    return out[:N_NODES]


# speedup vs baseline: 4.5449x; 1.1634x over previous
import functools

import jax
import jax.numpy as jnp
from jax import lax
from jax.experimental import pallas as pl
from jax.experimental.pallas import tpu as pltpu
from jax.experimental.pallas import tpu_sc as plsc

N_NODES = 100000
D = 128
NT = 513
NI = 256
K = 10

NC = 2
NS = 16
NW = NC * NS

B = 32
SUB = 8
G = B * SUB
NG = (N_NODES + G - 1) // G
N_PAD = NG * G
EXTRA = NG - (NG // NW) * NW


def _sc_body(table_hbm, idx_hbm, out_hbm, idx_v, rows_v, out_v,
             gsem0, gsem1, osem0, osem1):
    wid = lax.axis_index("s") * NC + lax.axis_index("c")
    ng = jnp.where(wid < EXTRA, NG // NW + 1, NG // NW)
    t_end = ng * SUB
    gsems = (gsem0, gsem1)
    osems = (osem0, osem1)

    def maybe_load_idx(t):
        r = t // SUB
        rp = lax.rem(r, 2)

        @pl.when(jnp.logical_and(lax.rem(t, SUB) == 0, t < t_end))
        def _():
            base = (wid + r * NW) * G
            for k in range(K):
                pltpu.sync_copy(idx_hbm.at[pl.ds(k * N_PAD + base, G)],
                                idx_v.at[rp, k])

    def gather_descs(t, p):
        r = t // SUB
        s = lax.rem(t, SUB)
        rp = lax.rem(r, 2)
        return [(table_hbm.at[idx_v.at[rp, k, pl.ds(s * B, B)]],
                 rows_v.at[p, k], gsems[p]) for k in range(K)]

    def fire(t, p):
        @pl.when(t < t_end)
        def _():
            for src, dst, sem in gather_descs(t, p):
                pltpu.async_copy(src, dst, sem)

    def wait_gathers(t, p):
        for src, dst, sem in gather_descs(t, p):
            pltpu.make_async_copy(src, dst, sem).wait()

    def compute_write(t, p):
        r = t // SUB
        s = lax.rem(t, SUB)
        sbase = (wid + r * NW) * G + s * B

        @pl.when(t >= 2)
        def _():
            pltpu.make_async_copy(out_v.at[p], out_hbm.at[pl.ds(sbase, B)],
                                  osems[p]).wait()

        def node_body(b, cc):
            for j in range(D // 16):
                acc = rows_v[p, 0, b, pl.ds(j * 16, 16)]
                for k in range(1, K):
                    acc = acc + rows_v[p, k, b, pl.ds(j * 16, 16)]
                out_v[p, b, pl.ds(j * 16, 16)] = acc
            return cc

        lax.fori_loop(0, B, node_body, 0)
        pltpu.async_copy(out_v.at[p], out_hbm.at[pl.ds(sbase, B)], osems[p])

    maybe_load_idx(jnp.int32(0))
    fire(jnp.int32(0), 0)

    def body(v, carry):
        t0 = 2 * v
        t1 = 2 * v + 1
        maybe_load_idx(t1)
        fire(t1, 1)
        wait_gathers(t0, 0)
        compute_write(t0, 0)
        maybe_load_idx(t1 + 1)
        fire(t1 + 1, 0)
        wait_gathers(t1, 1)
        compute_write(t1, 1)
        return carry

    lax.fori_loop(0, t_end // 2, body, 0)

    last0 = (wid + (t_end - 2) // SUB * NW) * G + lax.rem(t_end - 2, SUB) * B
    last1 = (wid + (t_end - 1) // SUB * NW) * G + lax.rem(t_end - 1, SUB) * B
    pltpu.make_async_copy(out_v.at[0], out_hbm.at[pl.ds(last0, B)],
                          osem0).wait()
    pltpu.make_async_copy(out_v.at[1], out_hbm.at[pl.ds(last1, B)],
                          osem1).wait()


def kernel(node_type, in_degree, out_degree, node_weight, in_degree_weight,
           out_degree_weight):
    table = jnp.concatenate(
        [node_weight, in_degree_weight, out_degree_weight], axis=0)
    idx = jnp.concatenate(
        [node_type.astype(jnp.int32),
         (in_degree.astype(jnp.int32) + NT)[:, None],
         (out_degree.astype(jnp.int32) + NT + NI)[:, None]], axis=1).T
    idx = jnp.pad(idx, ((0, 0), (0, N_PAD - N_NODES))).reshape(-1)

    mesh = plsc.VectorSubcoreMesh(core_axis_name="c", subcore_axis_name="s",
                                  num_cores=NC, num_subcores=NS)
    run = functools.partial(
        pl.kernel,
        out_type=jax.ShapeDtypeStruct((N_PAD, D), jnp.float32),
        mesh=mesh,
        scratch_types=[
            pltpu.VMEM((2, K, G), jnp.int32),
            pltpu.VMEM((2, K, B, D), jnp.float32),
            pltpu.VMEM((2, B, D), jnp.float32),
            pltpu.SemaphoreType.DMA,
            pltpu.SemaphoreType.DMA,
            pltpu.SemaphoreType.DMA,
            pltpu.SemaphoreType.DMA,
        ],
    )(_sc_body)
    out = run(table, idx)
    return out[:N_NODES]
